# baseline (speedup 1.0000x reference)
import jax
import jax.numpy as jnp
from jax.experimental import pallas as pl
from jax.experimental.pallas import tpu as pltpu

_NUM_NODES = 100
_TIME_STEPS = 64
_NUM_FILTERS = 32
_FILTER_SIZE = 4
_C = _NUM_NODES * _NUM_NODES
_OUT_POS = _TIME_STEPS - _FILTER_SIZE + 1
_GB = 4


def _tgcnn_kernel(gam_ref, x_ref, w_ref, o_ref):
    neg_gamma = -gam_ref[0, 0]
    dn = (((0,), (0,)), ((), ()))
    for g in range(_GB):
        xb = x_ref[g].reshape(_C, _TIME_STEPS)
        xv = jnp.where(xb != 0.0, jnp.exp(xb * neg_gamma), 0.0)
        acc = jax.lax.dot_general(w_ref[...], xv, dn,
                                  preferred_element_type=jnp.float32)
        o_ref[g] = (acc[0:32, 0:61] + acc[32:64, 1:62]
                    + acc[64:96, 2:63] + acc[96:128, 3:64])


def kernel(input_graphs, w, gammat):
    b = input_graphs.shape[0]
    wf = w.reshape(_C, _FILTER_SIZE * _NUM_FILTERS)
    gamma = 10.0 * jax.nn.sigmoid(gammat)

    out = pl.pallas_call(
        _tgcnn_kernel,
        grid=(b // _GB,),
        in_specs=[
            pl.BlockSpec((1, 1), lambda i: (0, 0), memory_space=pltpu.SMEM),
            pl.BlockSpec((_GB, _NUM_NODES, _NUM_NODES, _TIME_STEPS),
                         lambda i: (i, 0, 0, 0)),
            pl.BlockSpec((_C, _FILTER_SIZE * _NUM_FILTERS), lambda i: (0, 0)),
        ],
        out_specs=pl.BlockSpec((_GB, _NUM_FILTERS, _OUT_POS),
                               lambda i: (i, 0, 0)),
        out_shape=jax.ShapeDtypeStruct((b, _NUM_FILTERS, _OUT_POS), jnp.float32),
    )(gamma, input_graphs, wf)
    return out[:, :, None, :]


# speedup vs baseline: 1.0203x; 1.0203x over previous
import jax
import jax.numpy as jnp
from jax.experimental import pallas as pl
from jax.experimental.pallas import tpu as pltpu

_NUM_NODES = 100
_TIME_STEPS = 64
_NUM_FILTERS = 32
_FILTER_SIZE = 4
_C = _NUM_NODES * _NUM_NODES
_OUT_POS = _TIME_STEPS - _FILTER_SIZE + 1
_GB = 2


def _tgcnn_kernel(gam_ref, x_ref, w_ref, o_ref):
    neg_gamma = -gam_ref[0, 0]
    dn = (((0,), (0,)), ((), ()))
    for g in range(_GB):
        xb = x_ref[g].reshape(_C, _TIME_STEPS)
        xv = jnp.where(xb != 0.0, jnp.exp(xb * neg_gamma), 0.0)
        acc = jax.lax.dot_general(w_ref[...], xv, dn,
                                  preferred_element_type=jnp.float32)
        o_ref[g] = (acc[0:32, 0:61] + acc[32:64, 1:62]
                    + acc[64:96, 2:63] + acc[96:128, 3:64])


def kernel(input_graphs, w, gammat):
    b = input_graphs.shape[0]
    wf = w.reshape(_C, _FILTER_SIZE * _NUM_FILTERS)
    gamma = 10.0 * jax.nn.sigmoid(gammat)

    out = pl.pallas_call(
        _tgcnn_kernel,
        grid=(b // _GB,),
        in_specs=[
            pl.BlockSpec((1, 1), lambda i: (0, 0), memory_space=pltpu.SMEM),
            pl.BlockSpec((_GB, _NUM_NODES, _NUM_NODES, _TIME_STEPS),
                         lambda i: (i, 0, 0, 0)),
            pl.BlockSpec((_C, _FILTER_SIZE * _NUM_FILTERS), lambda i: (0, 0)),
        ],
        out_specs=pl.BlockSpec((_GB, _NUM_FILTERS, _OUT_POS),
                               lambda i: (i, 0, 0)),
        out_shape=jax.ShapeDtypeStruct((b, _NUM_FILTERS, _OUT_POS), jnp.float32),
    )(gamma, input_graphs, wf)
    return out[:, :, None, :]
